# Initial kernel scaffold; baseline (speedup 1.0000x reference)
import jax
import jax.numpy as jnp
from jax.experimental import pallas as pl


def kernel(input_ids, word_emb, pos_emb, ln_weight, ln_bias):
    raise NotImplementedError("write your pallas kernel here")


# speedup vs baseline: 2.3199x; 2.3199x over previous
import functools

import jax
import jax.numpy as jnp
from jax import lax
from jax.experimental import pallas as pl
from jax.experimental.pallas import tpu as pltpu
from jax.experimental.pallas import tpu_sc as plsc

_LANES = 16


def _rsqrt(x):
    i = lax.bitcast_convert_type(x, jnp.int32)
    i = jnp.int32(0x5F3759DF) - (i >> 1)
    y = lax.bitcast_convert_type(i, jnp.float32)
    for _ in range(3):
        y = y * (jnp.float32(1.5) - jnp.float32(0.5) * x * y * y)
    return y


_GATHER_DNUMS = lax.GatherDimensionNumbers(
    offset_dims=(), collapsed_slice_dims=(0,), start_index_map=(0,))


def _lane_perm(x, perm):
    return lax.gather(
        x, perm[:, None], _GATHER_DNUMS, slice_sizes=(1,),
        mode=lax.GatherScatterMode.PROMISE_IN_BOUNDS)


def _lane_sum(x):
    idx = lax.iota(jnp.int32, _LANES)
    for sh in (8, 4, 2, 1):
        x = x + _lane_perm(x, idx ^ sh)
    return x


@functools.lru_cache(maxsize=None)
def _build(B, L, H, V, P):
    assert H % _LANES == 0
    nh = H // _LANES

    chunk = L
    nsplit = 1
    while chunk > 128:
        nsplit *= 2
        assert L % nsplit == 0
        chunk = L // nsplit

    info = plsc.get_sparse_core_info()
    NC, NS = info.num_cores, info.num_subcores
    NW = NC * NS
    assert B % NW == 0
    seq_per_w = B // NW

    mesh = plsc.VectorSubcoreMesh(core_axis_name="c", subcore_axis_name="s")

    @functools.partial(
        pl.kernel,
        mesh=mesh,
        out_type=jax.ShapeDtypeStruct((B * L, H), jnp.float32),
        scratch_types=[
            pltpu.VMEM((nsplit, chunk), jnp.int32),
            pltpu.VMEM((L, H), jnp.float32),
            pltpu.VMEM((L, H), jnp.float32),
            pltpu.VMEM((H,), jnp.float32),
            pltpu.VMEM((H,), jnp.float32),
            pltpu.SemaphoreType.DMA,
        ],
    )
    def emb_kernel(ids_hbm, word_hbm, pos_hbm, w_hbm, b_hbm, out_hbm,
                   idx_v, rows_v, pos_v, w_v, b_v, sem):
        cid = lax.axis_index("c")
        sid = lax.axis_index("s")
        wid = sid * NC + cid

        pltpu.sync_copy(pos_hbm.at[pl.ds(0, L)], pos_v)
        pltpu.sync_copy(w_hbm, w_v)
        pltpu.sync_copy(b_hbm, b_v)

        inv_h = jnp.float32(1.0 / H)
        eps = jnp.float32(1e-12)

        def seq_body(g, carry):
            seq = wid * seq_per_w + g
            base = seq * L
            pltpu.sync_copy(ids_hbm.at[pl.ds(seq * nsplit, nsplit)], idx_v)
            copies = [
                pltpu.async_copy(
                    word_hbm.at[idx_v.at[k]],
                    rows_v.at[pl.ds(k * chunk, chunk)],
                    sem,
                )
                for k in range(nsplit)
            ]
            for c in copies:
                c.wait()

            def row_body(r, carry2):
                xs = [
                    rows_v[r, pl.ds(j * _LANES, _LANES)]
                    + pos_v[r, pl.ds(j * _LANES, _LANES)]
                    for j in range(nh)
                ]
                s = xs[0]
                for j in range(1, nh):
                    s = s + xs[j]
                mean = _lane_sum(s) * inv_h
                ds_ = [x - mean for x in xs]
                s2 = ds_[0] * ds_[0]
                for j in range(1, nh):
                    s2 = s2 + ds_[j] * ds_[j]
                var = _lane_sum(s2) * inv_h
                rinv = _rsqrt(var + eps)
                for j in range(nh):
                    sl = pl.ds(j * _LANES, _LANES)
                    rows_v[r, sl] = ds_[j] * rinv * w_v[sl] + b_v[sl]
                return carry2

            lax.fori_loop(0, L, row_body, 0)
            pltpu.sync_copy(rows_v, out_hbm.at[pl.ds(base, L)])
            return carry

        lax.fori_loop(0, seq_per_w, seq_body, 0)

    return emb_kernel, nsplit, chunk


def kernel(input_ids, word_emb, pos_emb, ln_weight, ln_bias):
    B, L = input_ids.shape
    V, H = word_emb.shape
    P = pos_emb.shape[0]
    emb_kernel, nsplit, chunk = _build(B, L, H, V, P)
    ids2d = input_ids.reshape(B * nsplit, chunk)
    out = emb_kernel(ids2d, word_emb, pos_emb, ln_weight, ln_bias)
    return out.reshape(B, L, H)


# speedup vs baseline: 7.5782x; 3.2666x over previous
import functools

import jax
import jax.numpy as jnp
from jax import lax
from jax.experimental import pallas as pl
from jax.experimental.pallas import tpu as pltpu
from jax.experimental.pallas import tpu_sc as plsc

_LANES = 16


def _rsqrt(x):
    i = lax.bitcast_convert_type(x, jnp.int32)
    i = jnp.int32(0x5F3759DF) - (i >> 1)
    y = lax.bitcast_convert_type(i, jnp.float32)
    for _ in range(3):
        y = y * (jnp.float32(1.5) - jnp.float32(0.5) * x * y * y)
    return y


_GATHER_DNUMS = lax.GatherDimensionNumbers(
    offset_dims=(), collapsed_slice_dims=(0,), start_index_map=(0,))


def _lane_perm(x, perm):
    return lax.gather(
        x, perm[:, None], _GATHER_DNUMS, slice_sizes=(1,),
        mode=lax.GatherScatterMode.PROMISE_IN_BOUNDS)


def _lane_sum(x):
    idx = lax.iota(jnp.int32, _LANES)
    for sh in (8, 4, 2, 1):
        x = x + _lane_perm(x, idx ^ sh)
    return x


@functools.lru_cache(maxsize=None)
def _build(B, L, H, V, P):
    assert H % _LANES == 0
    nh = H // _LANES

    chunk = L
    nsplit = 1
    while chunk > 128:
        nsplit *= 2
        assert L % nsplit == 0
        chunk = L // nsplit

    info = plsc.get_sparse_core_info()
    NC, NS = info.num_cores, info.num_subcores
    NW = NC * NS
    assert B % NW == 0
    seq_per_w = B // NW
    assert seq_per_w % 2 == 0

    mesh = plsc.VectorSubcoreMesh(core_axis_name="c", subcore_axis_name="s")

    @functools.partial(
        pl.kernel,
        mesh=mesh,
        out_type=jax.ShapeDtypeStruct((B * L, H), jnp.float32),
        scratch_types=[
            pltpu.VMEM((2, nsplit, chunk), jnp.int32),
            pltpu.VMEM((L, H), jnp.float32),
            pltpu.VMEM((L, H), jnp.float32),
            pltpu.VMEM((L, H), jnp.float32),
            pltpu.VMEM((H,), jnp.float32),
            pltpu.VMEM((H,), jnp.float32),
            pltpu.SemaphoreType.DMA,
            pltpu.SemaphoreType.DMA,
            pltpu.SemaphoreType.DMA,
            pltpu.SemaphoreType.DMA,
        ],
    )
    def emb_kernel(ids_hbm, word_hbm, pos_hbm, w_hbm, b_hbm, out_hbm,
                   idx_v, rows0_v, rows1_v, pos_v, w_v, b_v,
                   gsem0, gsem1, wsem0, wsem1):
        cid = lax.axis_index("c")
        sid = lax.axis_index("s")
        wid = sid * NC + cid

        rows = (rows0_v, rows1_v)
        gsem = (gsem0, gsem1)
        wsem = (wsem0, wsem1)

        pltpu.sync_copy(pos_hbm.at[pl.ds(0, L)], pos_v)
        pltpu.sync_copy(w_hbm, w_v)
        pltpu.sync_copy(b_hbm, b_v)

        inv_h = jnp.float32(1.0 / H)
        eps = jnp.float32(1e-12)
        seq0 = wid * seq_per_w

        def fire_gather(seq, b):
            pltpu.sync_copy(ids_hbm.at[pl.ds(seq * nsplit, nsplit)],
                            idx_v.at[b])
            for k in range(nsplit):
                pltpu.async_copy(
                    word_hbm.at[idx_v.at[b].at[k]],
                    rows[b].at[pl.ds(k * chunk, chunk)],
                    gsem[b],
                )

        def drain_gather(b):
            pltpu.make_async_copy(
                word_hbm.at[pl.ds(0, L)], rows[b], gsem[b]).wait()

        def drain_wb(b):
            pltpu.make_async_copy(
                word_hbm.at[pl.ds(0, L)], rows[b], wsem[b]).wait()

        fire_gather(seq0, 0)

        def ln_rows(b, r, wregs, bregs):
            rv = rows[b]
            for rr in (r, r + 1):
                xs = [
                    rv[rr, pl.ds(j * _LANES, _LANES)]
                    + pos_v[rr, pl.ds(j * _LANES, _LANES)]
                    for j in range(nh)
                ]
                s = xs[0]
                s2 = xs[0] * xs[0]
                for j in range(1, nh):
                    s = s + xs[j]
                    s2 = s2 + xs[j] * xs[j]
                mean = _lane_sum(s) * inv_h
                var = _lane_sum(s2) * inv_h - mean * mean
                rinv = _rsqrt(var + eps)
                for j in range(nh):
                    sl = pl.ds(j * _LANES, _LANES)
                    rv[rr, sl] = (xs[j] - mean) * rinv * wregs[j] + bregs[j]

        def pair_body(i, carry):
            for b in (0, 1):
                g = i * 2 + b
                seq = seq0 + g
                nxt_ok = jnp.int32(g) < jnp.int32(seq_per_w - 1)

                @pl.when(jnp.logical_and(nxt_ok, jnp.int32(g) >= 1))
                def _():
                    drain_wb(b ^ 1)

                @pl.when(nxt_ok)
                def _():
                    fire_gather(seq + 1, b ^ 1)

                drain_gather(b)

                wb0 = tuple(w_v[pl.ds(j * _LANES, _LANES)] for j in range(nh)) \
                    + tuple(b_v[pl.ds(j * _LANES, _LANES)] for j in range(nh))

                def row_body(r2, carry2):
                    ln_rows(b, r2 * 2, carry2[:nh], carry2[nh:])
                    return carry2

                lax.fori_loop(0, L // 2, row_body, wb0)
                pltpu.async_copy(rows[b], out_hbm.at[pl.ds(seq * L, L)],
                                 wsem[b])
            return carry

        lax.fori_loop(0, seq_per_w // 2, pair_body, 0)
        drain_wb(0)
        drain_wb(1)

    return emb_kernel, nsplit, chunk


def kernel(input_ids, word_emb, pos_emb, ln_weight, ln_bias):
    B, L = input_ids.shape
    V, H = word_emb.shape
    P = pos_emb.shape[0]
    emb_kernel, nsplit, chunk = _build(B, L, H, V, P)
    ids2d = input_ids.reshape(B * nsplit, chunk)
    out = emb_kernel(ids2d, word_emb, pos_emb, ln_weight, ln_bias)
    return out.reshape(B, L, H)
